# Initial kernel scaffold; baseline (speedup 1.0000x reference)
import functools

import jax
import jax.numpy as jnp
from jax import lax
from jax.experimental import pallas as pl
from jax.experimental.pallas import tpu as pltpu
from jax.experimental.pallas import tpu_sc as plsc

N_NODES = 10000
NPAD = 10240
N_EDGES = 320000
D_IN = 128
D_H1 = 128
D_H2 = 64

NC = 2
NS = 16
NW = NC * NS
EPW = N_EDGES // NW
CHUNK = 80
NCHUNK = EPW // CHUNK
ZONE = NPAD // NS

_mesh = lambda: plsc.VectorSubcoreMesh(core_axis_name="c", subcore_axis_name="s")


HW = 8


def _hist_body(dst_hbm, out_hbm, dstv, obuf, zbuf, acc_sh, sem):
    c = lax.axis_index("c")
    s = lax.axis_index("s")
    lane = lax.iota(jnp.int32, 16)
    one_pat = jnp.where(lane % HW == 0, 1.0, 0.0).astype(jnp.float32)
    zero_pat = jnp.zeros((16,), jnp.float32)
    for i in range(CHUNK * HW // 16):
        obuf[pl.ds(2 * i, 2), :] = one_pat.reshape(2, HW)
        zbuf[pl.ds(2 * i, 2), :] = zero_pat.reshape(2, HW)
    for i in range(ZONE // CHUNK):
        pltpu.sync_copy(zbuf, acc_sh.at[pl.ds(s * ZONE + i * CHUNK, CHUNK)])
    plsc.subcore_barrier()

    base = (c * NS + s) * EPW

    def step(k, carry):
        eb = pl.multiple_of(base + k * CHUNK, 8)
        pltpu.sync_copy(dst_hbm.at[pl.ds(eb, CHUNK)], dstv)
        pltpu.sync_copy(obuf, acc_sh.at[dstv], add=True)
        return carry

    lax.fori_loop(0, NCHUNK, step, 0)
    plsc.subcore_barrier()
    for i in range(ZONE // CHUNK):
        off = s * ZONE + i * CHUNK
        pltpu.sync_copy(acc_sh.at[pl.ds(off, CHUNK)], zbuf)
        pltpu.sync_copy(zbuf, out_hbm.at[c, pl.ds(off, CHUNK)])


def _make_hist():
    return pl.kernel(
        _hist_body,
        out_type=jax.ShapeDtypeStruct((NC, NPAD, HW), jnp.float32),
        mesh=_mesh(),
        scratch_types=[
            pltpu.VMEM((CHUNK,), jnp.int32),
            pltpu.VMEM((CHUNK, HW), jnp.float32),
            pltpu.VMEM((CHUNK, HW), jnp.float32),
            pltpu.VMEM_SHARED((NPAD, HW), jnp.float32),
            pltpu.SemaphoreType.DMA,
        ],
    )


def _agg_body(d, g_hbm, src_hbm, dst_hbm, out_hbm, srcv, dstv, rows, acc_sh, sem):
    c = lax.axis_index("c")
    s = lax.axis_index("s")
    zero_pat = jnp.zeros((16,), jnp.float32)

    def zstep(i, carry):
        for j in range(d // 16):
            rows[i, pl.ds(j * 16, 16)] = zero_pat
        return carry

    lax.fori_loop(0, CHUNK, zstep, 0)
    for i in range(ZONE // CHUNK):
        pltpu.sync_copy(rows, acc_sh.at[pl.ds(s * ZONE + i * CHUNK, CHUNK)])
    plsc.subcore_barrier()

    base = (c * NS + s) * EPW

    def step(k, carry):
        eb = pl.multiple_of(base + k * CHUNK, 8)
        pltpu.sync_copy(src_hbm.at[pl.ds(eb, CHUNK)], srcv)
        pltpu.sync_copy(dst_hbm.at[pl.ds(eb, CHUNK)], dstv)
        pltpu.async_copy(g_hbm.at[srcv], rows, sem).wait()
        pltpu.sync_copy(rows, acc_sh.at[dstv], add=True)
        return carry

    lax.fori_loop(0, NCHUNK, step, 0)
    plsc.subcore_barrier()
    for i in range(ZONE // CHUNK):
        off = s * ZONE + i * CHUNK
        pltpu.sync_copy(acc_sh.at[pl.ds(off, CHUNK)], rows)
        pltpu.sync_copy(rows, out_hbm.at[c, pl.ds(off, CHUNK)])


def _make_agg(d):
    return pl.kernel(
        functools.partial(_agg_body, d),
        out_type=jax.ShapeDtypeStruct((NC, NPAD, d), jnp.float32),
        mesh=_mesh(),
        scratch_types=[
            pltpu.VMEM((CHUNK,), jnp.int32),
            pltpu.VMEM((CHUNK,), jnp.int32),
            pltpu.VMEM((CHUNK, d), jnp.float32),
            pltpu.VMEM_SHARED((NPAD, d), jnp.float32),
            pltpu.SemaphoreType.DMA,
        ],
    )


BLK = 512
GRID = NPAD // BLK


def _tc_a_body(x_ref, w_ref, h0_ref, h1_ref, g_ref, dis_ref):
    deg = 1.0 + h0_ref[:, 0:1] + h1_ref[:, 0:1]
    dis = lax.rsqrt(deg)
    h = jnp.dot(x_ref[...], w_ref[...], preferred_element_type=jnp.float32)
    g_ref[...] = h * dis
    dis_ref[...] = dis


def _tc_b_body(a0_ref, a1_ref, g_ref, dis_ref, b_ref, w_ref, out_ref):
    dis = dis_ref[...]
    o1 = jnp.maximum(dis * (a0_ref[...] + a1_ref[...] + g_ref[...]) + b_ref[...], 0.0)
    out_ref[...] = dis * jnp.dot(o1, w_ref[...], preferred_element_type=jnp.float32)


def _tc_c_body(a0_ref, a1_ref, g_ref, dis_ref, b_ref, out_ref):
    dis = dis_ref[...]
    out_ref[...] = jnp.maximum(
        dis * (a0_ref[...] + a1_ref[...] + g_ref[...]) + b_ref[...], 0.0)


def _row_spec(d):
    return pl.BlockSpec((BLK, d), lambda i: (i, 0))


def _full_spec(r, c):
    return pl.BlockSpec((r, c), lambda i: (0, 0))


_tc_a = pl.pallas_call(
    _tc_a_body,
    grid=(GRID,),
    in_specs=[_row_spec(D_IN), _full_spec(D_IN, D_H1), _row_spec(HW), _row_spec(HW)],
    out_specs=[_row_spec(D_H1), _row_spec(1)],
    out_shape=[jax.ShapeDtypeStruct((NPAD, D_H1), jnp.float32),
               jax.ShapeDtypeStruct((NPAD, 1), jnp.float32)],
)

_tc_b = pl.pallas_call(
    _tc_b_body,
    grid=(GRID,),
    in_specs=[_row_spec(D_H1), _row_spec(D_H1), _row_spec(D_H1), _row_spec(1),
              _full_spec(1, D_H1), _full_spec(D_H1, D_H2)],
    out_specs=_row_spec(D_H2),
    out_shape=jax.ShapeDtypeStruct((NPAD, D_H2), jnp.float32),
)

_tc_c = pl.pallas_call(
    _tc_c_body,
    grid=(GRID,),
    in_specs=[_row_spec(D_H2), _row_spec(D_H2), _row_spec(D_H2), _row_spec(1),
              _full_spec(1, D_H2)],
    out_specs=_row_spec(D_H2),
    out_shape=jax.ShapeDtypeStruct((NPAD, D_H2), jnp.float32),
)

_hist = _make_hist()
_agg1 = _make_agg(D_H1)
_agg2 = _make_agg(D_H2)


def kernel(x, edge_index, W1, b1, W2, b2):
    ei = edge_index.astype(jnp.int32)
    src, dst = ei[0], ei[1]
    x_pad = jnp.pad(x, ((0, NPAD - N_NODES), (0, 0)))

    hist = _hist(dst)
    g1, dis = _tc_a(x_pad, W1, hist[0], hist[1])
    acc1 = _agg1(g1, src, dst)
    g2 = _tc_b(acc1[0], acc1[1], g1, dis, b1.reshape(1, -1), W2)
    acc2 = _agg2(g2, src, dst)
    out = _tc_c(acc2[0], acc2[1], g2, dis, b2.reshape(1, -1))
    return out[:N_NODES]


# speedup vs baseline: 13.0560x; 13.0560x over previous
import functools

import jax
import jax.numpy as jnp
from jax import lax
from jax.experimental import pallas as pl
from jax.experimental.pallas import tpu as pltpu
from jax.experimental.pallas import tpu_sc as plsc

N_NODES = 10000
NPAD = 10240
N_EDGES = 320000
D_IN = 128
D_H1 = 128
D_H2 = 64

NC = 2
NS = 16
NW = NC * NS
EPW = N_EDGES // NW
CHUNK = 80
NCHUNK = EPW // CHUNK
ZONE = NPAD // NS

_mesh = lambda: plsc.VectorSubcoreMesh(core_axis_name="c", subcore_axis_name="s")


HW = 8


def _hist_body(dst_hbm, ones_hbm, zeros_hbm, out_hbm, dstv, obuf, zbuf, acc_sh, sem):
    c = lax.axis_index("c")
    s = lax.axis_index("s")
    pltpu.sync_copy(ones_hbm, obuf)
    pltpu.sync_copy(zeros_hbm, zbuf)
    for i in range(ZONE // CHUNK):
        pltpu.sync_copy(zbuf, acc_sh.at[pl.ds(s * ZONE + i * CHUNK, CHUNK)])
    plsc.subcore_barrier()

    base = (c * NS + s) * EPW

    def step(k, carry):
        eb = pl.multiple_of(base + k * CHUNK, 8)
        pltpu.sync_copy(dst_hbm.at[pl.ds(eb, CHUNK)], dstv)
        pltpu.sync_copy(obuf, acc_sh.at[dstv], add=True)
        return carry

    lax.fori_loop(0, NCHUNK, step, 0)
    plsc.subcore_barrier()
    for i in range(ZONE // CHUNK):
        off = s * ZONE + i * CHUNK
        pltpu.sync_copy(acc_sh.at[pl.ds(off, CHUNK)], zbuf)
        pltpu.sync_copy(zbuf, out_hbm.at[c, pl.ds(off, CHUNK)])


def _make_hist():
    return pl.kernel(
        _hist_body,
        out_type=jax.ShapeDtypeStruct((NC, NPAD, HW), jnp.float32),
        mesh=_mesh(),
        compiler_params=pltpu.CompilerParams(use_tc_tiling_on_sc=False),
        scratch_types=[
            pltpu.VMEM((CHUNK,), jnp.int32),
            pltpu.VMEM((CHUNK, HW), jnp.float32),
            pltpu.VMEM((CHUNK, HW), jnp.float32),
            pltpu.VMEM_SHARED((NPAD, HW), jnp.float32),
            pltpu.SemaphoreType.DMA,
        ],
    )


def _agg_body(d, g_hbm, src_hbm, dst_hbm, out_hbm, srcv, dstv, rows, acc_sh, sem):
    c = lax.axis_index("c")
    s = lax.axis_index("s")
    zero_pat = jnp.zeros((16,), jnp.float32)

    def zstep(i, carry):
        for j in range(d // 16):
            rows[i, pl.ds(j * 16, 16)] = zero_pat
        return carry

    lax.fori_loop(0, CHUNK, zstep, 0)
    for i in range(ZONE // CHUNK):
        pltpu.sync_copy(rows, acc_sh.at[pl.ds(s * ZONE + i * CHUNK, CHUNK)])
    plsc.subcore_barrier()

    base = (c * NS + s) * EPW

    def step(k, carry):
        eb = pl.multiple_of(base + k * CHUNK, 8)
        pltpu.sync_copy(src_hbm.at[pl.ds(eb, CHUNK)], srcv)
        pltpu.sync_copy(dst_hbm.at[pl.ds(eb, CHUNK)], dstv)
        pltpu.async_copy(g_hbm.at[srcv], rows, sem).wait()
        pltpu.sync_copy(rows, acc_sh.at[dstv], add=True)
        return carry

    lax.fori_loop(0, NCHUNK, step, 0)
    plsc.subcore_barrier()
    for i in range(ZONE // CHUNK):
        off = s * ZONE + i * CHUNK
        pltpu.sync_copy(acc_sh.at[pl.ds(off, CHUNK)], rows)
        pltpu.sync_copy(rows, out_hbm.at[c, pl.ds(off, CHUNK)])


def _make_agg(d):
    return pl.kernel(
        functools.partial(_agg_body, d),
        out_type=jax.ShapeDtypeStruct((NC, NPAD, d), jnp.float32),
        mesh=_mesh(),
        compiler_params=pltpu.CompilerParams(use_tc_tiling_on_sc=False),
        scratch_types=[
            pltpu.VMEM((CHUNK,), jnp.int32),
            pltpu.VMEM((CHUNK,), jnp.int32),
            pltpu.VMEM((CHUNK, d), jnp.float32),
            pltpu.VMEM_SHARED((NPAD, d), jnp.float32),
            pltpu.SemaphoreType.DMA,
        ],
    )


BLK = 512
GRID = NPAD // BLK


def _tc_a_body(x_ref, w_ref, h0_ref, h1_ref, g_ref, dis_ref):
    deg = 1.0 + h0_ref[:, 0:1] + h1_ref[:, 0:1]
    dis = lax.rsqrt(deg)
    h = jnp.dot(x_ref[...], w_ref[...], preferred_element_type=jnp.float32)
    g_ref[...] = h * dis
    dis_ref[...] = dis


def _tc_b_body(a0_ref, a1_ref, g_ref, dis_ref, b_ref, w_ref, out_ref):
    dis = dis_ref[...]
    o1 = jnp.maximum(dis * (a0_ref[...] + a1_ref[...] + g_ref[...]) + b_ref[...], 0.0)
    out_ref[...] = dis * jnp.dot(o1, w_ref[...], preferred_element_type=jnp.float32)


def _tc_c_body(a0_ref, a1_ref, g_ref, dis_ref, b_ref, out_ref):
    dis = dis_ref[...]
    out_ref[...] = jnp.maximum(
        dis * (a0_ref[...] + a1_ref[...] + g_ref[...]) + b_ref[...], 0.0)


def _row_spec(d):
    return pl.BlockSpec((BLK, d), lambda i: (i, 0))


def _full_spec(r, c):
    return pl.BlockSpec((r, c), lambda i: (0, 0))


_tc_a = pl.pallas_call(
    _tc_a_body,
    grid=(GRID,),
    in_specs=[_row_spec(D_IN), _full_spec(D_IN, D_H1), _row_spec(HW), _row_spec(HW)],
    out_specs=[_row_spec(D_H1), _row_spec(1)],
    out_shape=[jax.ShapeDtypeStruct((NPAD, D_H1), jnp.float32),
               jax.ShapeDtypeStruct((NPAD, 1), jnp.float32)],
)

_tc_b = pl.pallas_call(
    _tc_b_body,
    grid=(GRID,),
    in_specs=[_row_spec(D_H1), _row_spec(D_H1), _row_spec(D_H1), _row_spec(1),
              _full_spec(1, D_H1), _full_spec(D_H1, D_H2)],
    out_specs=_row_spec(D_H2),
    out_shape=jax.ShapeDtypeStruct((NPAD, D_H2), jnp.float32),
)

_tc_c = pl.pallas_call(
    _tc_c_body,
    grid=(GRID,),
    in_specs=[_row_spec(D_H2), _row_spec(D_H2), _row_spec(D_H2), _row_spec(1),
              _full_spec(1, D_H2)],
    out_specs=_row_spec(D_H2),
    out_shape=jax.ShapeDtypeStruct((NPAD, D_H2), jnp.float32),
)

_hist = _make_hist()
_agg1 = _make_agg(D_H1)
_agg2 = _make_agg(D_H2)


def kernel(x, edge_index, W1, b1, W2, b2):
    ei = edge_index.astype(jnp.int32)
    src, dst = ei[0], ei[1]
    x_pad = jnp.pad(x, ((0, NPAD - N_NODES), (0, 0)))

    ones8 = jnp.tile(jnp.eye(1, HW, dtype=jnp.float32), (CHUNK, 1))
    zeros8 = jnp.zeros((CHUNK, HW), jnp.float32)
    hist = _hist(dst, ones8, zeros8)
    g1, dis = _tc_a(x_pad, W1, hist[0], hist[1])
    acc1 = _agg1(g1, src, dst)
    g2 = _tc_b(acc1[0], acc1[1], g1, dis, b1.reshape(1, -1), W2)
    acc2 = _agg2(g2, src, dst)
    out = _tc_c(acc2[0], acc2[1], g2, dis, b2.reshape(1, -1))
    return out[:N_NODES]
